# Initial kernel scaffold; baseline (speedup 1.0000x reference)
import jax
import jax.numpy as jnp
from jax.experimental import pallas as pl


def kernel(atomic_number, position, edge_index, W1, b1, W2, b2, W3, b3, W4, b4):
    raise NotImplementedError("write your pallas kernel here")


# speedup vs baseline: 8.2380x; 8.2380x over previous
import jax
import jax.numpy as jnp
from jax import lax
from jax.experimental import pallas as pl
from jax.experimental.pallas import tpu as pltpu
from jax.experimental.pallas import tpu_sc as plsc

N = 10000
NPAD = 10240
E = 320000
F = 64
CH = 128
EPAD = 327680
NCHUNK = EPAD // CH
NC, NS = 2, 16
STRIPE = NPAD // (NC * NS)
ZROWS = NPAD // NS
KB = 16
NBUF = 2

_mesh = plsc.VectorSubcoreMesh(core_axis_name="c", subcore_axis_name="s")


DW = 16


def _deg_body(srcr, dstr, zeros2, ones, out, degS, degD, sidx, didx, ones_v,
              semS, semD):
    cid = lax.axis_index("c")
    sid = lax.axis_index("s")
    wid = cid * NS + sid
    pltpu.sync_copy(ones, ones_v)
    pltpu.sync_copy(zeros2, degS.at[pl.ds(sid * ZROWS, ZROWS)])
    pltpu.sync_copy(zeros2, degD.at[pl.ds(sid * ZROWS, ZROWS)])
    plsc.subcore_barrier()

    nsup = NCHUNK // (NC * NS * KB)
    start = wid * (NCHUNK // (NC * NS))

    def outer(o, _):
        c0 = pl.multiple_of(start + o * KB, KB)
        pltpu.sync_copy(srcr.at[pl.ds(c0, KB)], sidx)
        pltpu.sync_copy(dstr.at[pl.ds(c0, KB)], didx)

        def fire(k, _):
            pltpu.async_copy(ones_v, degS.at[sidx.at[k]], semS, add=True)
            pltpu.async_copy(ones_v, degD.at[didx.at[k]], semD, add=True)
            return 0

        lax.fori_loop(0, KB, fire, 0, unroll=False)

        def drain(k, _):
            pltpu.make_async_copy(ones_v, degS.at[sidx.at[0]], semS).wait()
            pltpu.make_async_copy(ones_v, degD.at[didx.at[0]], semD).wait()
            return 0

        lax.fori_loop(0, KB, drain, 0, unroll=False)
        return 0

    lax.fori_loop(0, nsup, outer, 0, unroll=False)
    plsc.subcore_barrier()
    pltpu.sync_copy(degS.at[pl.ds(sid * ZROWS, ZROWS)],
                    out.at[cid, 0, pl.ds(sid * ZROWS, ZROWS)])
    pltpu.sync_copy(degD.at[pl.ds(sid * ZROWS, ZROWS)],
                    out.at[cid, 1, pl.ds(sid * ZROWS, ZROWS)])


def _deg_kernel(srcr, dstr, zeros2, ones):
    return pl.kernel(
        _deg_body,
        out_type=jax.ShapeDtypeStruct((NC, 2, NPAD, DW), jnp.float32),
        mesh=_mesh,
        compiler_params=pltpu.CompilerParams(use_tc_tiling_on_sc=False),
        scratch_types=[
            pltpu.VMEM_SHARED((NPAD, DW), jnp.float32),
            pltpu.VMEM_SHARED((NPAD, DW), jnp.float32),
            pltpu.VMEM((KB, CH), jnp.int32),
            pltpu.VMEM((KB, CH), jnp.int32),
            pltpu.VMEM((CH, DW), jnp.float32),
            pltpu.SemaphoreType.DMA,
            pltpu.SemaphoreType.DMA,
        ],
    )(srcr, dstr, zeros2, ones)


def _pass_body(table, srcr, dstr, zeros, out,
               acc, sidx, didx, rows0, rows1, gsem0, gsem1, ssem0, ssem1):
    cid = lax.axis_index("c")
    sid = lax.axis_index("s")
    wid = cid * NS + sid
    rows = (rows0, rows1)
    gsem = (gsem0, gsem1)
    ssem = (ssem0, ssem1)

    pltpu.sync_copy(zeros, acc.at[pl.ds(sid * ZROWS, ZROWS)])
    plsc.subcore_barrier()

    nsup = NCHUNK // (NS * KB)
    start = sid * (NCHUNK // NS)

    def outer(o, _):
        c0 = pl.multiple_of(start + o * KB, KB)
        pltpu.sync_copy(srcr.at[pl.ds(c0, KB)], sidx)
        pltpu.sync_copy(dstr.at[pl.ds(c0, KB)], didx)

        def inner(g, _):
            for s in range(NBUF):
                k = g * NBUF + s

                @pl.when(g > 0)
                def _wait_prev():
                    pltpu.make_async_copy(
                        rows[s], acc.at[didx.at[k]], ssem[s]).wait()

                pltpu.async_copy(table.at[sidx.at[k]], rows[s], gsem[s])
            for s in range(NBUF):
                k = g * NBUF + s
                pltpu.make_async_copy(
                    table.at[sidx.at[k]], rows[s], gsem[s]).wait()
                pltpu.async_copy(rows[s], acc.at[didx.at[k]], ssem[s],
                                 add=True)
            return 0

        lax.fori_loop(0, KB // NBUF, inner, 0, unroll=False)
        for s in range(NBUF):
            pltpu.make_async_copy(rows[s], acc.at[didx.at[s]], ssem[s]).wait()
        return 0

    lax.fori_loop(0, nsup, outer, 0, unroll=False)
    plsc.subcore_barrier()
    rs = wid * STRIPE
    pltpu.sync_copy(acc.at[pl.ds(rs, STRIPE)], out.at[pl.ds(rs, STRIPE)])


def _pass_kernel(table, srcr, dstr, zeros):
    return pl.kernel(
        _pass_body,
        out_type=jax.ShapeDtypeStruct((NPAD, F), jnp.float32),
        mesh=_mesh,
        compiler_params=pltpu.CompilerParams(use_tc_tiling_on_sc=False),
        scratch_types=[
            pltpu.VMEM_SHARED((NPAD, F), jnp.float32),
            pltpu.VMEM((KB, CH), jnp.int32),
            pltpu.VMEM((KB, CH), jnp.int32),
            pltpu.VMEM((CH, F), jnp.float32),
            pltpu.VMEM((CH, F), jnp.float32),
            pltpu.SemaphoreType.DMA,
            pltpu.SemaphoreType.DMA,
            pltpu.SemaphoreType.DMA,
            pltpu.SemaphoreType.DMA,
        ],
    )(table, srcr, dstr, zeros)


def _pre_body(at_ref, pos_ref, w1, b1, w2, b2, w3, b3, w4, b4, degp,
              t0_ref, m_ref, nsrc_ref, ndst_ref, cmat_ref):
    degS = degp[0, :, 0] + degp[2, :, 0]
    degD = degp[1, :, 0] + degp[3, :, 0]
    nsrc = lax.rsqrt(jnp.maximum(degS, 1.0))
    ndst = lax.rsqrt(jnp.maximum(degD, 1.0))
    nsrc_ref[...] = nsrc
    ndst_ref[...] = ndst
    m_ref[...] = nsrc * ndst

    w34 = w3[...] @ w4[...]
    w234 = w2[...] @ w34
    wc = w1[...] @ w234
    hx = jnp.concatenate([at_ref[...], pos_ref[...]], axis=1)
    y0 = hx @ wc
    t0 = y0 * nsrc[:N, None]
    t0_ref[...] = jnp.concatenate(
        [t0, jnp.zeros((NPAD - N, F), jnp.float32)], axis=0)

    c1 = b1[...].reshape(1, -1) @ w234
    c2 = b2[...].reshape(1, -1) @ w34
    c3 = b3[...].reshape(1, -1) @ w4[...]
    c4 = b4[...].reshape(1, -1)
    cmat_ref[...] = jnp.concatenate([c1, c2, c3, c4], axis=0)


def _pre_kernel(at, pos, w1, b1, w2, b2, w3, b3, w4, b4, degp):
    return pl.pallas_call(
        _pre_body,
        out_shape=(
            jax.ShapeDtypeStruct((NPAD, F), jnp.float32),
            jax.ShapeDtypeStruct((NPAD,), jnp.float32),
            jax.ShapeDtypeStruct((NPAD,), jnp.float32),
            jax.ShapeDtypeStruct((NPAD,), jnp.float32),
            jax.ShapeDtypeStruct((4, F), jnp.float32),
        ),
    )(at, pos, w1, b1, w2, b2, w3, b3, w4, b4, degp)


def _comb_body(raw_ref, m_ref, nsrc_ref, ck_ref, out_ref):
    out_ref[...] = (raw_ref[...] * m_ref[...][:, None]
                    + nsrc_ref[...][:, None] * ck_ref[...])


def _comb_kernel(raw, m, nsrc, ck):
    return pl.pallas_call(
        _comb_body,
        out_shape=jax.ShapeDtypeStruct((NPAD, F), jnp.float32),
    )(raw, m, nsrc, ck)


def _final_body(raw_ref, ndst_ref, c4_ref, out_ref):
    h4 = (raw_ref[pl.ds(0, N), :] * ndst_ref[...][:N, None] + c4_ref[...])
    out_ref[...] = jnp.mean(h4.reshape(N // 5, 5, F), axis=1)


def _final_kernel(raw, ndst, c4):
    return pl.pallas_call(
        _final_body,
        out_shape=jax.ShapeDtypeStruct((N // 5, F), jnp.float32),
    )(raw, ndst, c4)


def kernel(atomic_number, position, edge_index, W1, b1, W2, b2, W3, b3, W4, b4):
    src = edge_index[0]
    dst = edge_index[1]
    npd = EPAD - E
    pad_ids = (N + (jnp.arange(npd, dtype=jnp.int32) % (NPAD - N)))
    srcr = jnp.concatenate([src, pad_ids]).reshape(NCHUNK, CH)
    dstr = jnp.concatenate([dst, pad_ids]).reshape(NCHUNK, CH)
    zeros = jnp.zeros((ZROWS, F), jnp.float32)
    zeros2 = jnp.zeros((ZROWS, DW), jnp.float32)
    ones = jnp.ones((CH, DW), jnp.float32)

    degp = _deg_kernel(srcr, dstr, zeros2, ones).reshape(4, NPAD, DW)
    t0, m, nsrc, ndst, cmat = _pre_kernel(
        atomic_number, position, W1, b1, W2, b2, W3, b3, W4, b4, degp)

    t = t0
    for k in range(3):
        raw = _pass_kernel(t, srcr, dstr, zeros)
        t = _comb_kernel(raw, m, nsrc, cmat[k:k + 1])
    raw = _pass_kernel(t, srcr, dstr, zeros)
    return _final_kernel(raw, ndst, cmat[3:4])


# speedup vs baseline: 14.5634x; 1.7678x over previous
import jax
import jax.numpy as jnp
from jax import lax
from jax.experimental import pallas as pl
from jax.experimental.pallas import tpu as pltpu
from jax.experimental.pallas import tpu_sc as plsc

N = 10000
NPAD = 10240
E = 320000
F = 64
CH = 128
EPAD = 327680
NCHUNK = EPAD // CH
NC, NS = 2, 16
STRIPE = NPAD // (NC * NS)
ZROWS = NPAD // NS
KB = 16
NBUF = 4

_mesh = plsc.VectorSubcoreMesh(core_axis_name="c", subcore_axis_name="s")


DW = 16


def _deg_body(srcr, dstr, zeros2, ones, out, degS, degD, sidx, didx, ones_v,
              semS, semD):
    cid = lax.axis_index("c")
    sid = lax.axis_index("s")
    wid = cid * NS + sid
    pltpu.sync_copy(ones, ones_v)
    pltpu.sync_copy(zeros2, degS.at[pl.ds(sid * ZROWS, ZROWS)])
    pltpu.sync_copy(zeros2, degD.at[pl.ds(sid * ZROWS, ZROWS)])
    plsc.subcore_barrier()

    nsup = NCHUNK // (NC * NS * KB)
    start = wid * (NCHUNK // (NC * NS))

    def outer(o, _):
        c0 = pl.multiple_of(start + o * KB, KB)
        pltpu.sync_copy(srcr.at[pl.ds(c0, KB)], sidx)
        pltpu.sync_copy(dstr.at[pl.ds(c0, KB)], didx)

        def fire(k, _):
            pltpu.async_copy(ones_v, degS.at[sidx.at[k]], semS, add=True)
            pltpu.async_copy(ones_v, degD.at[didx.at[k]], semD, add=True)
            return 0

        lax.fori_loop(0, KB, fire, 0, unroll=False)

        def drain(k, _):
            pltpu.make_async_copy(ones_v, degS.at[sidx.at[0]], semS).wait()
            pltpu.make_async_copy(ones_v, degD.at[didx.at[0]], semD).wait()
            return 0

        lax.fori_loop(0, KB, drain, 0, unroll=False)
        return 0

    lax.fori_loop(0, nsup, outer, 0, unroll=False)
    plsc.subcore_barrier()
    pltpu.sync_copy(degS.at[pl.ds(sid * ZROWS, ZROWS)],
                    out.at[cid, 0, pl.ds(sid * ZROWS, ZROWS)])
    pltpu.sync_copy(degD.at[pl.ds(sid * ZROWS, ZROWS)],
                    out.at[cid, 1, pl.ds(sid * ZROWS, ZROWS)])


def _deg_kernel(srcr, dstr, zeros2, ones):
    return pl.kernel(
        _deg_body,
        out_type=jax.ShapeDtypeStruct((NC, 2, NPAD, DW), jnp.float32),
        mesh=_mesh,
        compiler_params=pltpu.CompilerParams(use_tc_tiling_on_sc=False),
        scratch_types=[
            pltpu.VMEM_SHARED((NPAD, DW), jnp.float32),
            pltpu.VMEM_SHARED((NPAD, DW), jnp.float32),
            pltpu.VMEM((KB, CH), jnp.int32),
            pltpu.VMEM((KB, CH), jnp.int32),
            pltpu.VMEM((CH, DW), jnp.float32),
            pltpu.SemaphoreType.DMA,
            pltpu.SemaphoreType.DMA,
        ],
    )(srcr, dstr, zeros2, ones)


def _pass_body(table, srcr, dstr, zeros, out,
               acc, sidx, didx, rows0, rows1, rows2, rows3,
               gsem0, gsem1, gsem2, gsem3, ssem0, ssem1, ssem2, ssem3):
    cid = lax.axis_index("c")
    sid = lax.axis_index("s")
    wid = cid * NS + sid
    rows = (rows0, rows1, rows2, rows3)
    gsem = (gsem0, gsem1, gsem2, gsem3)
    ssem = (ssem0, ssem1, ssem2, ssem3)

    pltpu.sync_copy(zeros, acc.at[pl.ds(sid * ZROWS, ZROWS)])
    plsc.subcore_barrier()

    nsup = NCHUNK // (NC * NS * KB)
    start = wid * (NCHUNK // (NC * NS))

    def outer(o, _):
        c0 = pl.multiple_of(start + o * KB, KB)
        pltpu.sync_copy(srcr.at[pl.ds(c0, KB)], sidx)
        pltpu.sync_copy(dstr.at[pl.ds(c0, KB)], didx)

        def inner(g, _):
            for s in range(NBUF):
                k = g * NBUF + s

                @pl.when(g > 0)
                def _wait_prev():
                    pltpu.make_async_copy(
                        rows[s], acc.at[didx.at[k]], ssem[s]).wait()

                pltpu.async_copy(table.at[sidx.at[k]], rows[s], gsem[s])
            for s in range(NBUF):
                k = g * NBUF + s
                pltpu.make_async_copy(
                    table.at[sidx.at[k]], rows[s], gsem[s]).wait()
                pltpu.async_copy(rows[s], acc.at[didx.at[k]], ssem[s],
                                 add=True)
            return 0

        lax.fori_loop(0, KB // NBUF, inner, 0, unroll=False)
        for s in range(NBUF):
            pltpu.make_async_copy(rows[s], acc.at[didx.at[s]], ssem[s]).wait()
        return 0

    lax.fori_loop(0, nsup, outer, 0, unroll=False)
    plsc.subcore_barrier()
    rs = sid * ZROWS
    pltpu.sync_copy(acc.at[pl.ds(rs, ZROWS)], out.at[cid, pl.ds(rs, ZROWS)])


def _pass_kernel(table, srcr, dstr, zeros):
    return pl.kernel(
        _pass_body,
        out_type=jax.ShapeDtypeStruct((NC, NPAD, F), jnp.float32),
        mesh=_mesh,
        compiler_params=pltpu.CompilerParams(use_tc_tiling_on_sc=False),
        scratch_types=[
            pltpu.VMEM_SHARED((NPAD, F), jnp.float32),
            pltpu.VMEM((KB, CH), jnp.int32),
            pltpu.VMEM((KB, CH), jnp.int32),
            pltpu.VMEM((CH, F), jnp.float32),
            pltpu.VMEM((CH, F), jnp.float32),
            pltpu.VMEM((CH, F), jnp.float32),
            pltpu.VMEM((CH, F), jnp.float32),
            pltpu.SemaphoreType.DMA,
            pltpu.SemaphoreType.DMA,
            pltpu.SemaphoreType.DMA,
            pltpu.SemaphoreType.DMA,
            pltpu.SemaphoreType.DMA,
            pltpu.SemaphoreType.DMA,
            pltpu.SemaphoreType.DMA,
            pltpu.SemaphoreType.DMA,
        ],
    )(table, srcr, dstr, zeros)


def _pre_body(at_ref, pos_ref, w1, b1, w2, b2, w3, b3, w4, b4, degp,
              t0_ref, m_ref, nsrc_ref, ndst_ref, cmat_ref):
    degS = degp[0, :, 0] + degp[2, :, 0]
    degD = degp[1, :, 0] + degp[3, :, 0]
    nsrc = lax.rsqrt(jnp.maximum(degS, 1.0))
    ndst = lax.rsqrt(jnp.maximum(degD, 1.0))
    nsrc_ref[...] = nsrc
    ndst_ref[...] = ndst
    m_ref[...] = nsrc * ndst

    w34 = w3[...] @ w4[...]
    w234 = w2[...] @ w34
    wc = w1[...] @ w234
    hx = jnp.concatenate([at_ref[...], pos_ref[...]], axis=1)
    y0 = hx @ wc
    t0 = y0 * nsrc[:N, None]
    t0_ref[...] = jnp.concatenate(
        [t0, jnp.zeros((NPAD - N, F), jnp.float32)], axis=0)

    c1 = b1[...].reshape(1, -1) @ w234
    c2 = b2[...].reshape(1, -1) @ w34
    c3 = b3[...].reshape(1, -1) @ w4[...]
    c4 = b4[...].reshape(1, -1)
    cmat_ref[...] = jnp.concatenate([c1, c2, c3, c4], axis=0)


def _pre_kernel(at, pos, w1, b1, w2, b2, w3, b3, w4, b4, degp):
    return pl.pallas_call(
        _pre_body,
        out_shape=(
            jax.ShapeDtypeStruct((NPAD, F), jnp.float32),
            jax.ShapeDtypeStruct((NPAD,), jnp.float32),
            jax.ShapeDtypeStruct((NPAD,), jnp.float32),
            jax.ShapeDtypeStruct((NPAD,), jnp.float32),
            jax.ShapeDtypeStruct((4, F), jnp.float32),
        ),
    )(at, pos, w1, b1, w2, b2, w3, b3, w4, b4, degp)


def _comb_body(raw_ref, m_ref, nsrc_ref, ck_ref, out_ref):
    raw = raw_ref[0] + raw_ref[1]
    out_ref[...] = (raw * m_ref[...][:, None]
                    + nsrc_ref[...][:, None] * ck_ref[...])


def _comb_kernel(raw, m, nsrc, ck):
    return pl.pallas_call(
        _comb_body,
        out_shape=jax.ShapeDtypeStruct((NPAD, F), jnp.float32),
    )(raw, m, nsrc, ck)


def _final_body(raw_ref, ndst_ref, c4_ref, out_ref):
    raw = raw_ref[0, pl.ds(0, N), :] + raw_ref[1, pl.ds(0, N), :]
    h4 = (raw * ndst_ref[...][:N, None] + c4_ref[...])
    out_ref[...] = jnp.mean(h4.reshape(N // 5, 5, F), axis=1)


def _final_kernel(raw, ndst, c4):
    return pl.pallas_call(
        _final_body,
        out_shape=jax.ShapeDtypeStruct((N // 5, F), jnp.float32),
    )(raw, ndst, c4)


def kernel(atomic_number, position, edge_index, W1, b1, W2, b2, W3, b3, W4, b4):
    src = edge_index[0]
    dst = edge_index[1]
    npd = EPAD - E
    pad_ids = (N + (jnp.arange(npd, dtype=jnp.int32) % (NPAD - N)))
    srcr = jnp.concatenate([src, pad_ids]).reshape(NCHUNK, CH)
    dstr = jnp.concatenate([dst, pad_ids]).reshape(NCHUNK, CH)
    zeros = jnp.zeros((ZROWS, F), jnp.float32)
    zeros2 = jnp.zeros((ZROWS, DW), jnp.float32)
    ones = jnp.ones((CH, DW), jnp.float32)

    degp = _deg_kernel(srcr, dstr, zeros2, ones).reshape(4, NPAD, DW)
    t0, m, nsrc, ndst, cmat = _pre_kernel(
        atomic_number, position, W1, b1, W2, b2, W3, b3, W4, b4, degp)

    t = t0
    for k in range(3):
        raw = _pass_kernel(t, srcr, dstr, zeros)
        t = _comb_kernel(raw, m, nsrc, cmat[k:k + 1])
    raw = _pass_kernel(t, srcr, dstr, zeros)
    return _final_kernel(raw, ndst, cmat[3:4])


# speedup vs baseline: 16.8155x; 1.1546x over previous
import jax
import jax.numpy as jnp
from jax import lax
from jax.experimental import pallas as pl
from jax.experimental.pallas import tpu as pltpu
from jax.experimental.pallas import tpu_sc as plsc

N = 10000
NPAD = 10240
E = 320000
F = 64
CH = 128
EPAD = 327680
NCHUNK = EPAD // CH
NC, NS = 2, 16
STRIPE = NPAD // (NC * NS)
ZROWS = NPAD // NS
KB = 16
NBUF = 6

_mesh = plsc.VectorSubcoreMesh(core_axis_name="c", subcore_axis_name="s")


DW = 16


def _deg_body(srcr, dstr, zeros2, ones, out, degS, degD, sidx, didx, ones_v,
              semS, semD):
    cid = lax.axis_index("c")
    sid = lax.axis_index("s")
    wid = cid * NS + sid
    pltpu.sync_copy(ones, ones_v)
    pltpu.sync_copy(zeros2, degS.at[pl.ds(sid * ZROWS, ZROWS)])
    pltpu.sync_copy(zeros2, degD.at[pl.ds(sid * ZROWS, ZROWS)])
    c0 = pl.multiple_of(wid * KPT, KPT)
    pltpu.sync_copy(srcr.at[pl.ds(c0, KPT)], sidx)
    pltpu.sync_copy(dstr.at[pl.ds(c0, KPT)], didx)
    plsc.subcore_barrier()

    W = 8
    for k in range(KPT):
        pltpu.async_copy(ones_v, degS.at[sidx.at[k]], semS, add=True)
        pltpu.async_copy(ones_v, degD.at[didx.at[k]], semD, add=True)
        if k >= W:
            pltpu.make_async_copy(ones_v, degS.at[sidx.at[0]], semS).wait()
            pltpu.make_async_copy(ones_v, degD.at[didx.at[0]], semD).wait()
    for k in range(W):
        pltpu.make_async_copy(ones_v, degS.at[sidx.at[0]], semS).wait()
        pltpu.make_async_copy(ones_v, degD.at[didx.at[0]], semD).wait()
    plsc.subcore_barrier()
    pltpu.sync_copy(degS.at[pl.ds(sid * ZROWS, ZROWS)],
                    out.at[cid, 0, pl.ds(sid * ZROWS, ZROWS)])
    pltpu.sync_copy(degD.at[pl.ds(sid * ZROWS, ZROWS)],
                    out.at[cid, 1, pl.ds(sid * ZROWS, ZROWS)])


def _deg_kernel(srcr, dstr, zeros2, ones):
    return pl.kernel(
        _deg_body,
        out_type=jax.ShapeDtypeStruct((NC, 2, NPAD, DW), jnp.float32),
        mesh=_mesh,
        compiler_params=pltpu.CompilerParams(use_tc_tiling_on_sc=False),
        scratch_types=[
            pltpu.VMEM_SHARED((NPAD, DW), jnp.float32),
            pltpu.VMEM_SHARED((NPAD, DW), jnp.float32),
            pltpu.VMEM((KPT, CH), jnp.int32),
            pltpu.VMEM((KPT, CH), jnp.int32),
            pltpu.VMEM((CH, DW), jnp.float32),
            pltpu.SemaphoreType.DMA,
            pltpu.SemaphoreType.DMA,
        ],
    )(srcr, dstr, zeros2, ones)


KPT = NCHUNK // (NC * NS)


def _pass_body(table, srcr, dstr, zeros, out,
               acc, sidx, didx, rows0, rows1, rows2, rows3, rows4, rows5,
               gsem0, gsem1, gsem2, gsem3, gsem4, gsem5,
               ssem0, ssem1, ssem2, ssem3, ssem4, ssem5):
    cid = lax.axis_index("c")
    sid = lax.axis_index("s")
    wid = cid * NS + sid
    rows = (rows0, rows1, rows2, rows3, rows4, rows5)
    gsem = (gsem0, gsem1, gsem2, gsem3, gsem4, gsem5)
    ssem = (ssem0, ssem1, ssem2, ssem3, ssem4, ssem5)

    c0 = pl.multiple_of(wid * KPT, KPT)
    pltpu.async_copy(srcr.at[pl.ds(c0, KPT)], sidx, gsem[0])
    pltpu.async_copy(dstr.at[pl.ds(c0, KPT)], didx, gsem[1])
    pltpu.sync_copy(zeros, acc.at[pl.ds(sid * ZROWS, ZROWS)])
    pltpu.make_async_copy(srcr.at[pl.ds(c0, KPT)], sidx, gsem[0]).wait()
    pltpu.make_async_copy(dstr.at[pl.ds(c0, KPT)], didx, gsem[1]).wait()
    plsc.subcore_barrier()

    for k in range(KPT):
        b = k % NBUF
        if k >= NBUF:
            pltpu.make_async_copy(
                rows[b], acc.at[didx.at[k - NBUF]], ssem[b]).wait()
        pltpu.async_copy(table.at[sidx.at[k]], rows[b], gsem[b])
        if k >= NBUF - 1:
            kp = k - (NBUF - 1)
            bp = kp % NBUF
            pltpu.make_async_copy(
                table.at[sidx.at[kp]], rows[bp], gsem[bp]).wait()
            pltpu.async_copy(rows[bp], acc.at[didx.at[kp]], ssem[bp],
                             add=True)
    for kp in range(KPT - NBUF + 1, KPT):
        bp = kp % NBUF
        pltpu.make_async_copy(table.at[sidx.at[kp]], rows[bp], gsem[bp]).wait()
        pltpu.async_copy(rows[bp], acc.at[didx.at[kp]], ssem[bp], add=True)
    for kp in range(KPT - NBUF, KPT):
        bp = kp % NBUF
        pltpu.make_async_copy(rows[bp], acc.at[didx.at[kp]], ssem[bp]).wait()

    plsc.subcore_barrier()
    rs = sid * ZROWS
    pltpu.sync_copy(acc.at[pl.ds(rs, ZROWS)], out.at[cid, pl.ds(rs, ZROWS)])


def _pass_kernel(table, srcr, dstr, zeros):
    return pl.kernel(
        _pass_body,
        out_type=jax.ShapeDtypeStruct((NC, NPAD, F), jnp.float32),
        mesh=_mesh,
        compiler_params=pltpu.CompilerParams(use_tc_tiling_on_sc=False),
        scratch_types=[
            pltpu.VMEM_SHARED((NPAD, F), jnp.float32),
            pltpu.VMEM((KPT, CH), jnp.int32),
            pltpu.VMEM((KPT, CH), jnp.int32),
            pltpu.VMEM((CH, F), jnp.float32),
            pltpu.VMEM((CH, F), jnp.float32),
            pltpu.VMEM((CH, F), jnp.float32),
            pltpu.VMEM((CH, F), jnp.float32),
            pltpu.VMEM((CH, F), jnp.float32),
            pltpu.VMEM((CH, F), jnp.float32),
            pltpu.SemaphoreType.DMA,
            pltpu.SemaphoreType.DMA,
            pltpu.SemaphoreType.DMA,
            pltpu.SemaphoreType.DMA,
            pltpu.SemaphoreType.DMA,
            pltpu.SemaphoreType.DMA,
            pltpu.SemaphoreType.DMA,
            pltpu.SemaphoreType.DMA,
            pltpu.SemaphoreType.DMA,
            pltpu.SemaphoreType.DMA,
            pltpu.SemaphoreType.DMA,
            pltpu.SemaphoreType.DMA,
        ],
    )(table, srcr, dstr, zeros)


def _mm_body(at_ref, pos_ref, w1, b1, w2, b2, w3, b3, w4, b4,
             y0_ref, cmat_ref):
    w34 = w3[...] @ w4[...]
    w234 = w2[...] @ w34
    wc = w1[...] @ w234
    hx = jnp.concatenate([at_ref[...], pos_ref[...]], axis=1)
    y0_ref[...] = hx @ wc
    c1 = b1[...].reshape(1, -1) @ w234
    c2 = b2[...].reshape(1, -1) @ w34
    c3 = b3[...].reshape(1, -1) @ w4[...]
    c4 = b4[...].reshape(1, -1)
    cmat_ref[...] = jnp.concatenate([c1, c2, c3, c4], axis=0)


def _mm_kernel(at, pos, w1, b1, w2, b2, w3, b3, w4, b4):
    return pl.pallas_call(
        _mm_body,
        out_shape=(
            jax.ShapeDtypeStruct((N, F), jnp.float32),
            jax.ShapeDtypeStruct((4, F), jnp.float32),
        ),
    )(at, pos, w1, b1, w2, b2, w3, b3, w4, b4)


def _pre_body(y0_ref, degp, t0_ref, m_ref, nsrc_ref, ndst_ref):
    degS = degp[0, :, 0] + degp[2, :, 0]
    degD = degp[1, :, 0] + degp[3, :, 0]
    nsrc = lax.rsqrt(jnp.maximum(degS, 1.0))
    ndst = lax.rsqrt(jnp.maximum(degD, 1.0))
    nsrc_ref[...] = nsrc
    ndst_ref[...] = ndst
    m_ref[...] = nsrc * ndst
    t0 = y0_ref[...] * nsrc[:N, None]
    t0_ref[...] = jnp.concatenate(
        [t0, jnp.zeros((NPAD - N, F), jnp.float32)], axis=0)


def _pre_kernel(y0, degp):
    return pl.pallas_call(
        _pre_body,
        out_shape=(
            jax.ShapeDtypeStruct((NPAD, F), jnp.float32),
            jax.ShapeDtypeStruct((NPAD,), jnp.float32),
            jax.ShapeDtypeStruct((NPAD,), jnp.float32),
            jax.ShapeDtypeStruct((NPAD,), jnp.float32),
        ),
    )(y0, degp)


def _comb_body(raw_ref, m_ref, nsrc_ref, ck_ref, out_ref):
    raw = raw_ref[0] + raw_ref[1]
    out_ref[...] = (raw * m_ref[...][:, None]
                    + nsrc_ref[...][:, None] * ck_ref[...])


def _comb_kernel(raw, m, nsrc, ck):
    return pl.pallas_call(
        _comb_body,
        out_shape=jax.ShapeDtypeStruct((NPAD, F), jnp.float32),
    )(raw, m, nsrc, ck)


def _final_body(raw_ref, ndst_ref, c4_ref, out_ref):
    raw = raw_ref[0, pl.ds(0, N), :] + raw_ref[1, pl.ds(0, N), :]
    h4 = (raw * ndst_ref[...][:N, None] + c4_ref[...])
    out_ref[...] = jnp.mean(h4.reshape(N // 5, 5, F), axis=1)


def _final_kernel(raw, ndst, c4):
    return pl.pallas_call(
        _final_body,
        out_shape=jax.ShapeDtypeStruct((N // 5, F), jnp.float32),
    )(raw, ndst, c4)


def kernel(atomic_number, position, edge_index, W1, b1, W2, b2, W3, b3, W4, b4):
    src = edge_index[0]
    dst = edge_index[1]
    npd = EPAD - E
    pad_ids = (N + (jnp.arange(npd, dtype=jnp.int32) % (NPAD - N)))
    srcr = jnp.concatenate([src, pad_ids]).reshape(NCHUNK, CH)
    dstr = jnp.concatenate([dst, pad_ids]).reshape(NCHUNK, CH)
    zeros = jnp.zeros((ZROWS, F), jnp.float32)
    zeros2 = jnp.zeros((ZROWS, DW), jnp.float32)
    ones = jnp.ones((CH, DW), jnp.float32)

    degp = _deg_kernel(srcr, dstr, zeros2, ones).reshape(4, NPAD, DW)
    y0, cmat = _mm_kernel(
        atomic_number, position, W1, b1, W2, b2, W3, b3, W4, b4)
    t0, m, nsrc, ndst = _pre_kernel(y0, degp)

    t = t0
    for k in range(3):
        raw = _pass_kernel(t, srcr, dstr, zeros)
        t = _comb_kernel(raw, m, nsrc, cmat[k:k + 1])
    raw = _pass_kernel(t, srcr, dstr, zeros)
    return _final_kernel(raw, ndst, cmat[3:4])
